# baseline (speedup 1.0000x reference)
import functools

import jax
import jax.numpy as jnp
from jax import lax
from jax.experimental import pallas as pl
from jax.experimental.pallas import tpu as pltpu
from jax.experimental.pallas import tpu_sc as plsc

N = 10000
D = 128
E = 320000
NC = 2
NS = 16
NW = NC * NS
CHUNK = 64
PASSES = 2
CPP0 = 80
CPP1 = 80
CPPG = CPP1 + 2
NPAD = 10240
RPT = NPAD // NS
BLK = 1024


def _mesh():
    return plsc.VectorSubcoreMesh(
        core_axis_name="c", subcore_axis_name="s", num_cores=NC, num_subcores=NS
    )


@functools.partial(
    pl.kernel,
    out_type=jax.ShapeDtypeStruct((NW, 1, RPT), jnp.float32),
    mesh=_mesh(),
    scratch_types=[
        pltpu.VMEM_SHARED((NPAD,), jnp.float32),
        pltpu.VMEM((PASSES * CPPG, CHUNK), jnp.int32),
        pltpu.VMEM((CHUNK,), jnp.float32),
        pltpu.SemaphoreType.DMA,
    ],
)
def _sc_degree(zeros1_hbm, dst_hbm, out_hbm, acc, dstv, ones_v, sem):
    cid = lax.axis_index("c")
    sid = lax.axis_index("s")
    wid = cid * NS + sid
    pltpu.sync_copy(zeros1_hbm.at[pl.ds(sid * RPT, RPT)],
                    acc.at[pl.ds(sid * RPT, RPT)])
    for i in range(CHUNK // 16):
        ones_v[pl.ds(i * 16, 16)] = jnp.ones((16,), jnp.float32)
    pltpu.sync_copy(dst_hbm.at[wid], dstv)
    plsc.subcore_barrier()
    descs = []
    for j in range(PASSES * CPPG):
        descs.append(pltpu.async_copy(ones_v, acc.at[dstv.at[j]], sem, add=True))
    for d in descs:
        d.wait()
    plsc.subcore_barrier()
    pltpu.sync_copy(acc.at[pl.ds(sid * RPT, RPT)], out_hbm.at[wid, 0])


@functools.partial(
    pl.kernel,
    out_type=jax.ShapeDtypeStruct((NC, NPAD, D), jnp.float32),
    mesh=_mesh(),
    scratch_types=[
        pltpu.VMEM_SHARED((NPAD, D), jnp.float32),
        pltpu.VMEM((CPPG, CHUNK), jnp.int32),
        pltpu.VMEM((CPPG, CHUNK), jnp.int32),
        pltpu.VMEM((CHUNK, D), jnp.float32),
        pltpu.VMEM((CHUNK, D), jnp.float32),
        pltpu.SemaphoreType.DMA,
        pltpu.SemaphoreType.DMA,
        pltpu.SemaphoreType.DMA,
    ],
)
def _sc_propagate(y_hbm, src_hbm, dst_hbm, zeros_hbm, out_hbm,
                  acc, srcv, dstv, buf0, buf1, gsem0, gsem1, ssem):
    cid = lax.axis_index("c")
    sid = lax.axis_index("s")
    wid = cid * NS + sid
    cpp = jnp.where(cid == 0, CPP0, CPP1)
    pltpu.sync_copy(zeros_hbm.at[pl.ds(sid * RPT, RPT)],
                    acc.at[pl.ds(sid * RPT, RPT)])
    plsc.subcore_barrier()

    for p in range(PASSES):
        pltpu.sync_copy(src_hbm.at[wid, p], srcv)
        pltpu.sync_copy(dst_hbm.at[wid, p], dstv)

        pltpu.async_copy(y_hbm.at[srcv.at[0]], buf0, gsem0)
        pltpu.async_copy(y_hbm.at[srcv.at[1]], buf1, gsem1)

        def body(i, carry):
            j0 = 2 * i
            pltpu.make_async_copy(y_hbm.at[srcv.at[j0]], buf0, gsem0).wait()
            pltpu.async_copy(buf0, acc.at[dstv.at[j0]], ssem, add=True).wait()
            pltpu.async_copy(y_hbm.at[srcv.at[j0 + 2]], buf0, gsem0)
            pltpu.make_async_copy(y_hbm.at[srcv.at[j0 + 1]], buf1, gsem1).wait()
            pltpu.async_copy(buf1, acc.at[dstv.at[j0 + 1]], ssem, add=True).wait()
            pltpu.async_copy(y_hbm.at[srcv.at[j0 + 3]], buf1, gsem1)
            return carry

        lax.fori_loop(0, cpp // 2, body, 0)
        pltpu.make_async_copy(y_hbm.at[srcv.at[cpp]], buf0, gsem0).wait()
        pltpu.make_async_copy(y_hbm.at[srcv.at[cpp + 1]], buf1, gsem1).wait()

    plsc.subcore_barrier()
    pltpu.sync_copy(acc.at[pl.ds(sid * RPT, RPT)],
                    out_hbm.at[cid, pl.ds(sid * RPT, RPT)])


def _dis(deg_ref):
    deg = deg_ref[0, :] + deg_ref[1, :] + 1.0
    return lax.rsqrt(deg)


def _tc1_body(deg_ref, x_ref, w_ref, y_ref):
    dis = _dis(deg_ref)
    y_ref[...] = dis[:, None] * jnp.dot(
        x_ref[...], w_ref[...], preferred_element_type=jnp.float32)


def _tc2_body(deg_ref, p_ref, y_ref, b_ref, w_ref, o_ref):
    dis = _dis(deg_ref)
    agg = p_ref[0] + p_ref[1] + y_ref[...]
    h = jnp.maximum(dis[:, None] * agg + b_ref[...], 0.0)
    o_ref[...] = dis[:, None] * jnp.dot(
        h, w_ref[...], preferred_element_type=jnp.float32)


def _tc3_body(deg_ref, p_ref, y_ref, b_ref, wl_ref, bl_ref, o_ref):
    dis = _dis(deg_ref)
    agg = p_ref[0] + p_ref[1] + y_ref[...]
    h = jnp.maximum(dis[:, None] * agg + b_ref[...], 0.0)
    o_ref[...] = jnp.dot(h, wl_ref[...],
                         preferred_element_type=jnp.float32) + bl_ref[...]


_GRID = NPAD // BLK
_deg_spec = pl.BlockSpec((NC, BLK), lambda i: (0, i))
_row_spec = pl.BlockSpec((BLK, D), lambda i: (i, 0))
_p_spec = pl.BlockSpec((NC, BLK, D), lambda i: (0, i, 0))
_w_spec = pl.BlockSpec((D, D), lambda i: (0, 0))
_b_spec = pl.BlockSpec((1, D), lambda i: (0, 0))
_wl_spec = pl.BlockSpec((D, 1), lambda i: (0, 0))
_bl_spec = pl.BlockSpec((1, 1), lambda i: (0, 0))

_tc1 = pl.pallas_call(
    _tc1_body, grid=(_GRID,),
    in_specs=[_deg_spec, _row_spec, _w_spec],
    out_specs=_row_spec,
    out_shape=jax.ShapeDtypeStruct((NPAD, D), jnp.float32),
)
_tc2 = pl.pallas_call(
    _tc2_body, grid=(_GRID,),
    in_specs=[_deg_spec, _p_spec, _row_spec, _b_spec, _w_spec],
    out_specs=_row_spec,
    out_shape=jax.ShapeDtypeStruct((NPAD, D), jnp.float32),
)
_tc3 = pl.pallas_call(
    _tc3_body, grid=(_GRID,),
    in_specs=[_deg_spec, _p_spec, _row_spec, _b_spec, _wl_spec, _bl_spec],
    out_specs=pl.BlockSpec((BLK, 1), lambda i: (i, 0)),
    out_shape=jax.ShapeDtypeStruct((NPAD, 1), jnp.float32),
)


def _layout_edges(idx):
    n0 = NS * PASSES * CPP0 * CHUNK
    n1 = NS * PASSES * CPP1 * CHUNK
    idx = jnp.concatenate([idx, jnp.full((n0 + n1 - E,), N, jnp.int32)])
    part0 = idx[:n0].reshape(NS, PASSES, CPP0, CHUNK)
    part1 = idx[n0:].reshape(NS, PASSES, CPP1, CHUNK)
    out = jnp.full((NW, PASSES, CPPG, CHUNK), N, jnp.int32)
    out = out.at[:NS, :, :CPP0].set(part0)
    out = out.at[NS:, :, :CPP1].set(part1)
    return out


def kernel(x, edge_index, W1, b1, W2, b2, Wl, bl):
    ei = edge_index.astype(jnp.int32)
    src4d = _layout_edges(ei[0])
    dst4d = _layout_edges(ei[1])
    dst_deg = dst4d.reshape(NW, PASSES * CPPG, CHUNK)
    zeros1 = jnp.zeros((NPAD,), jnp.float32)
    zeros2 = jnp.zeros((NPAD, D), jnp.float32)
    xp = jnp.pad(x, ((0, NPAD - N), (0, 0)))
    b1r = b1.reshape(1, D)
    b2r = b2.reshape(1, D)
    blr = bl.reshape(1, 1)

    degp = _sc_degree(zeros1, dst_deg).reshape(NC, NPAD)
    y1 = _tc1(degp, xp, W1)
    p1 = _sc_propagate(y1, src4d, dst4d, zeros2)
    y2 = _tc2(degp, p1, y1, b1r, W2)
    p2 = _sc_propagate(y2, src4d, dst4d, zeros2)
    out = _tc3(degp, p2, y2, b2r, Wl, blr)
    return out[:N, 0]


# speedup vs baseline: 1.2277x; 1.2277x over previous
import functools

import jax
import jax.numpy as jnp
from jax import lax
from jax.experimental import pallas as pl
from jax.experimental.pallas import tpu as pltpu
from jax.experimental.pallas import tpu_sc as plsc

N = 10000
D = 128
E = 320000
NC = 2
NS = 16
NW = NC * NS
CHUNK = 64
PASSES = 2
CPP0 = 98
CPP1 = 60
CPPG = max(CPP0, CPP1) + 2
NPAD = 10240
RPT = NPAD // NS
BLK = 1024


def _mesh():
    return plsc.VectorSubcoreMesh(
        core_axis_name="c", subcore_axis_name="s", num_cores=NC, num_subcores=NS
    )


@functools.partial(
    pl.kernel,
    out_type=jax.ShapeDtypeStruct((NW, 1, RPT), jnp.float32),
    mesh=_mesh(),
    scratch_types=[
        pltpu.VMEM_SHARED((NPAD,), jnp.float32),
        pltpu.VMEM((PASSES * CPPG, CHUNK), jnp.int32),
        pltpu.VMEM((CHUNK,), jnp.float32),
        pltpu.SemaphoreType.DMA,
    ],
)
def _sc_degree(zeros1_hbm, dst_hbm, out_hbm, acc, dstv, ones_v, sem):
    cid = lax.axis_index("c")
    sid = lax.axis_index("s")
    wid = cid * NS + sid
    pltpu.sync_copy(zeros1_hbm.at[pl.ds(sid * RPT, RPT)],
                    acc.at[pl.ds(sid * RPT, RPT)])
    for i in range(CHUNK // 16):
        ones_v[pl.ds(i * 16, 16)] = jnp.ones((16,), jnp.float32)
    pltpu.sync_copy(dst_hbm.at[wid], dstv)
    plsc.subcore_barrier()
    descs = []
    for j in range(PASSES * CPPG):
        descs.append(pltpu.async_copy(ones_v, acc.at[dstv.at[j]], sem, add=True))
    for d in descs:
        d.wait()
    plsc.subcore_barrier()
    pltpu.sync_copy(acc.at[pl.ds(sid * RPT, RPT)], out_hbm.at[wid, 0])


@functools.partial(
    pl.kernel,
    out_type=jax.ShapeDtypeStruct((NC, NPAD, D), jnp.float32),
    mesh=_mesh(),
    scratch_types=[
        pltpu.VMEM_SHARED((NPAD, D), jnp.float32),
        pltpu.VMEM((CPPG, CHUNK), jnp.int32),
        pltpu.VMEM((CPPG, CHUNK), jnp.int32),
        pltpu.VMEM((CHUNK, D), jnp.float32),
        pltpu.VMEM((CHUNK, D), jnp.float32),
        pltpu.SemaphoreType.DMA,
        pltpu.SemaphoreType.DMA,
        pltpu.SemaphoreType.DMA,
    ],
)
def _sc_propagate(y_hbm, src_hbm, dst_hbm, zeros_hbm, out_hbm,
                  acc, srcv, dstv, buf0, buf1, gsem0, gsem1, ssem):
    cid = lax.axis_index("c")
    sid = lax.axis_index("s")
    wid = cid * NS + sid
    cpp = jnp.where(cid == 0, CPP0, CPP1)
    pltpu.sync_copy(zeros_hbm.at[pl.ds(sid * RPT, RPT)],
                    acc.at[pl.ds(sid * RPT, RPT)])
    plsc.subcore_barrier()

    for p in range(PASSES):
        pltpu.sync_copy(src_hbm.at[wid, p], srcv)
        pltpu.sync_copy(dst_hbm.at[wid, p], dstv)

        pltpu.async_copy(y_hbm.at[srcv.at[0]], buf0, gsem0)
        pltpu.async_copy(y_hbm.at[srcv.at[1]], buf1, gsem1)

        def body(i, carry):
            j0 = 2 * i
            pltpu.make_async_copy(y_hbm.at[srcv.at[j0]], buf0, gsem0).wait()
            pltpu.async_copy(buf0, acc.at[dstv.at[j0]], ssem, add=True).wait()
            pltpu.async_copy(y_hbm.at[srcv.at[j0 + 2]], buf0, gsem0)
            pltpu.make_async_copy(y_hbm.at[srcv.at[j0 + 1]], buf1, gsem1).wait()
            pltpu.async_copy(buf1, acc.at[dstv.at[j0 + 1]], ssem, add=True).wait()
            pltpu.async_copy(y_hbm.at[srcv.at[j0 + 3]], buf1, gsem1)
            return carry

        lax.fori_loop(0, cpp // 2, body, 0)
        pltpu.make_async_copy(y_hbm.at[srcv.at[cpp]], buf0, gsem0).wait()
        pltpu.make_async_copy(y_hbm.at[srcv.at[cpp + 1]], buf1, gsem1).wait()

    plsc.subcore_barrier()
    pltpu.sync_copy(acc.at[pl.ds(sid * RPT, RPT)],
                    out_hbm.at[cid, pl.ds(sid * RPT, RPT)])


def _dis(deg_ref):
    deg = deg_ref[0, :] + deg_ref[1, :] + 1.0
    return lax.rsqrt(deg)


def _tc1_body(deg_ref, x_ref, w_ref, y_ref):
    dis = _dis(deg_ref)
    y_ref[...] = dis[:, None] * jnp.dot(
        x_ref[...], w_ref[...], preferred_element_type=jnp.float32)


def _tc2_body(deg_ref, p_ref, y_ref, b_ref, w_ref, o_ref):
    dis = _dis(deg_ref)
    agg = p_ref[0] + p_ref[1] + y_ref[...]
    h = jnp.maximum(dis[:, None] * agg + b_ref[...], 0.0)
    o_ref[...] = dis[:, None] * jnp.dot(
        h, w_ref[...], preferred_element_type=jnp.float32)


def _tc3_body(deg_ref, p_ref, y_ref, b_ref, wl_ref, bl_ref, o_ref):
    dis = _dis(deg_ref)
    agg = p_ref[0] + p_ref[1] + y_ref[...]
    h = jnp.maximum(dis[:, None] * agg + b_ref[...], 0.0)
    o_ref[...] = jnp.dot(h, wl_ref[...],
                         preferred_element_type=jnp.float32) + bl_ref[...]


_GRID = NPAD // BLK
_deg_spec = pl.BlockSpec((NC, BLK), lambda i: (0, i))
_row_spec = pl.BlockSpec((BLK, D), lambda i: (i, 0))
_p_spec = pl.BlockSpec((NC, BLK, D), lambda i: (0, i, 0))
_w_spec = pl.BlockSpec((D, D), lambda i: (0, 0))
_b_spec = pl.BlockSpec((1, D), lambda i: (0, 0))
_wl_spec = pl.BlockSpec((D, 1), lambda i: (0, 0))
_bl_spec = pl.BlockSpec((1, 1), lambda i: (0, 0))

_tc1 = pl.pallas_call(
    _tc1_body, grid=(_GRID,),
    in_specs=[_deg_spec, _row_spec, _w_spec],
    out_specs=_row_spec,
    out_shape=jax.ShapeDtypeStruct((NPAD, D), jnp.float32),
)
_tc2 = pl.pallas_call(
    _tc2_body, grid=(_GRID,),
    in_specs=[_deg_spec, _p_spec, _row_spec, _b_spec, _w_spec],
    out_specs=_row_spec,
    out_shape=jax.ShapeDtypeStruct((NPAD, D), jnp.float32),
)
_tc3 = pl.pallas_call(
    _tc3_body, grid=(_GRID,),
    in_specs=[_deg_spec, _p_spec, _row_spec, _b_spec, _wl_spec, _bl_spec],
    out_specs=pl.BlockSpec((BLK, 1), lambda i: (i, 0)),
    out_shape=jax.ShapeDtypeStruct((NPAD, 1), jnp.float32),
)


def _layout_edges(idx):
    n0 = NS * PASSES * CPP0 * CHUNK
    n1 = NS * PASSES * CPP1 * CHUNK
    idx = jnp.concatenate([idx, jnp.full((n0 + n1 - E,), N, jnp.int32)])
    part0 = idx[:n0].reshape(NS, PASSES, CPP0, CHUNK)
    part1 = idx[n0:].reshape(NS, PASSES, CPP1, CHUNK)
    out = jnp.full((NW, PASSES, CPPG, CHUNK), N, jnp.int32)
    out = out.at[:NS, :, :CPP0].set(part0)
    out = out.at[NS:, :, :CPP1].set(part1)
    return out


def kernel(x, edge_index, W1, b1, W2, b2, Wl, bl):
    ei = edge_index.astype(jnp.int32)
    src4d = _layout_edges(ei[0])
    dst4d = _layout_edges(ei[1])
    dst_deg = dst4d.reshape(NW, PASSES * CPPG, CHUNK)
    zeros1 = jnp.zeros((NPAD,), jnp.float32)
    zeros2 = jnp.zeros((NPAD, D), jnp.float32)
    xp = jnp.pad(x, ((0, NPAD - N), (0, 0)))
    b1r = b1.reshape(1, D)
    b2r = b2.reshape(1, D)
    blr = bl.reshape(1, 1)

    degp = _sc_degree(zeros1, dst_deg).reshape(NC, NPAD)
    y1 = _tc1(degp, xp, W1)
    p1 = _sc_propagate(y1, src4d, dst4d, zeros2)
    y2 = _tc2(degp, p1, y1, b1r, W2)
    p2 = _sc_propagate(y2, src4d, dst4d, zeros2)
    out = _tc3(degp, p2, y2, b2r, Wl, blr)
    return out[:N, 0]


# speedup vs baseline: 3.6986x; 3.0126x over previous
import functools

import jax
import jax.numpy as jnp
from jax import lax
from jax.experimental import pallas as pl
from jax.experimental.pallas import tpu as pltpu
from jax.experimental.pallas import tpu_sc as plsc

N = 10000
D = 128
E = 320000
NC = 2
NS = 16
NW = NC * NS
CHUNK = 64
PASSES = 2
CPP0 = 90
CPP1 = 68
CPPG = max(CPP0, CPP1) + 2
NPAD = 10240
RPT = NPAD // NS
BLK = 1024


def _mesh():
    return plsc.VectorSubcoreMesh(
        core_axis_name="c", subcore_axis_name="s", num_cores=NC, num_subcores=NS
    )


@functools.partial(
    pl.kernel,
    out_type=jax.ShapeDtypeStruct((NW, 1, RPT), jnp.float32),
    mesh=_mesh(),
    scratch_types=[
        pltpu.VMEM_SHARED((NPAD,), jnp.float32),
        pltpu.VMEM((PASSES * CPPG, CHUNK), jnp.int32),
        pltpu.VMEM((CHUNK,), jnp.float32),
        pltpu.SemaphoreType.DMA,
    ],
)
def _sc_degree(zeros1_hbm, dst_hbm, out_hbm, acc, dstv, ones_v, sem):
    cid = lax.axis_index("c")
    sid = lax.axis_index("s")
    wid = cid * NS + sid
    pltpu.sync_copy(zeros1_hbm.at[pl.ds(sid * RPT, RPT)],
                    acc.at[pl.ds(sid * RPT, RPT)])
    for i in range(CHUNK // 16):
        ones_v[pl.ds(i * 16, 16)] = jnp.ones((16,), jnp.float32)
    pltpu.sync_copy(dst_hbm.at[wid], dstv)
    plsc.subcore_barrier()
    descs = []
    for j in range(PASSES * CPPG):
        descs.append(pltpu.async_copy(ones_v, acc.at[dstv.at[j]], sem, add=True))
    for d in descs:
        d.wait()
    plsc.subcore_barrier()
    pltpu.sync_copy(acc.at[pl.ds(sid * RPT, RPT)], out_hbm.at[wid, 0])


@functools.partial(
    pl.kernel,
    out_type=jax.ShapeDtypeStruct((NC, NPAD, D), jnp.float32),
    mesh=_mesh(),
    scratch_types=[
        pltpu.VMEM_SHARED((NPAD, D), jnp.float32),
        pltpu.VMEM((CPPG, CHUNK), jnp.int32),
        pltpu.VMEM((CPPG, CHUNK), jnp.int32),
        pltpu.VMEM((CHUNK, D), jnp.float32),
        pltpu.VMEM((CHUNK, D), jnp.float32),
        pltpu.SemaphoreType.DMA,
        pltpu.SemaphoreType.DMA,
        pltpu.SemaphoreType.DMA,
    ],
)
def _sc_propagate(y_hbm, src_hbm, dst_hbm, zeros_hbm, out_hbm,
                  acc, srcv, dstv, buf0, buf1, gsem0, gsem1, ssem):
    cid = lax.axis_index("c")
    sid = lax.axis_index("s")
    wid = cid * NS + sid
    cpp = jnp.where(cid == 0, CPP0, CPP1)
    pltpu.sync_copy(zeros_hbm.at[pl.ds(sid * RPT, RPT)],
                    acc.at[pl.ds(sid * RPT, RPT)])
    plsc.subcore_barrier()

    for p in range(PASSES):
        pltpu.sync_copy(src_hbm.at[wid, p], srcv)
        pltpu.sync_copy(dst_hbm.at[wid, p], dstv)

        pltpu.async_copy(y_hbm.at[srcv.at[0]], buf0, gsem0)
        pltpu.async_copy(y_hbm.at[srcv.at[1]], buf1, gsem1)

        def body(i, carry):
            j0 = 2 * i
            pltpu.make_async_copy(y_hbm.at[srcv.at[j0]], buf0, gsem0).wait()
            pltpu.async_copy(buf0, acc.at[dstv.at[j0]], ssem, add=True).wait()
            pltpu.async_copy(y_hbm.at[srcv.at[j0 + 2]], buf0, gsem0)
            pltpu.make_async_copy(y_hbm.at[srcv.at[j0 + 1]], buf1, gsem1).wait()
            pltpu.async_copy(buf1, acc.at[dstv.at[j0 + 1]], ssem, add=True).wait()
            pltpu.async_copy(y_hbm.at[srcv.at[j0 + 3]], buf1, gsem1)
            return carry

        lax.fori_loop(0, cpp // 2, body, 0)
        pltpu.make_async_copy(y_hbm.at[srcv.at[cpp]], buf0, gsem0).wait()
        pltpu.make_async_copy(y_hbm.at[srcv.at[cpp + 1]], buf1, gsem1).wait()

    plsc.subcore_barrier()
    pltpu.sync_copy(acc.at[pl.ds(sid * RPT, RPT)],
                    out_hbm.at[cid, pl.ds(sid * RPT, RPT)])


def _dis(deg_ref):
    deg = deg_ref[0, :] + deg_ref[1, :] + 1.0
    return lax.rsqrt(deg)


def _tc1_body(deg_ref, x_ref, w_ref, y_ref):
    dis = _dis(deg_ref)
    y_ref[...] = dis[:, None] * jnp.dot(
        x_ref[...], w_ref[...], preferred_element_type=jnp.float32)


def _tc2_body(deg_ref, p_ref, y_ref, b_ref, w_ref, o_ref):
    dis = _dis(deg_ref)
    agg = p_ref[0] + p_ref[1] + y_ref[...]
    h = jnp.maximum(dis[:, None] * agg + b_ref[...], 0.0)
    o_ref[...] = dis[:, None] * jnp.dot(
        h, w_ref[...], preferred_element_type=jnp.float32)


def _tc3_body(deg_ref, p_ref, y_ref, b_ref, wl_ref, bl_ref, o_ref):
    dis = _dis(deg_ref)
    agg = p_ref[0] + p_ref[1] + y_ref[...]
    h = jnp.maximum(dis[:, None] * agg + b_ref[...], 0.0)
    o_ref[...] = jnp.dot(h, wl_ref[...],
                         preferred_element_type=jnp.float32) + bl_ref[...]


_GRID = NPAD // BLK
_deg_spec = pl.BlockSpec((NC, BLK), lambda i: (0, i))
_row_spec = pl.BlockSpec((BLK, D), lambda i: (i, 0))
_p_spec = pl.BlockSpec((NC, BLK, D), lambda i: (0, i, 0))
_w_spec = pl.BlockSpec((D, D), lambda i: (0, 0))
_b_spec = pl.BlockSpec((1, D), lambda i: (0, 0))
_wl_spec = pl.BlockSpec((D, 1), lambda i: (0, 0))
_bl_spec = pl.BlockSpec((1, 1), lambda i: (0, 0))

_tc1 = pl.pallas_call(
    _tc1_body, grid=(_GRID,),
    in_specs=[_deg_spec, _row_spec, _w_spec],
    out_specs=_row_spec,
    out_shape=jax.ShapeDtypeStruct((NPAD, D), jnp.float32),
)
_tc2 = pl.pallas_call(
    _tc2_body, grid=(_GRID,),
    in_specs=[_deg_spec, _p_spec, _row_spec, _b_spec, _w_spec],
    out_specs=_row_spec,
    out_shape=jax.ShapeDtypeStruct((NPAD, D), jnp.float32),
)
_tc3 = pl.pallas_call(
    _tc3_body, grid=(_GRID,),
    in_specs=[_deg_spec, _p_spec, _row_spec, _b_spec, _wl_spec, _bl_spec],
    out_specs=pl.BlockSpec((BLK, 1), lambda i: (i, 0)),
    out_shape=jax.ShapeDtypeStruct((NPAD, 1), jnp.float32),
)


def _layout_edges(idx):
    n0 = NS * PASSES * CPP0 * CHUNK
    n1 = NS * PASSES * CPP1 * CHUNK
    pad = N + (jnp.arange(n0 + n1 - E, dtype=jnp.int32) % (NPAD - N))
    idx = jnp.concatenate([idx, pad])
    part0 = idx[:n0].reshape(NS, PASSES, CPP0, CHUNK)
    part1 = idx[n0:].reshape(NS, PASSES, CPP1, CHUNK)
    out = N + (jnp.arange(NW * PASSES * CPPG * CHUNK, dtype=jnp.int32)
               % (NPAD - N)).reshape(NW, PASSES, CPPG, CHUNK)
    out = out.at[:NS, :, :CPP0].set(part0)
    out = out.at[NS:, :, :CPP1].set(part1)
    return out


def kernel(x, edge_index, W1, b1, W2, b2, Wl, bl):
    ei = edge_index.astype(jnp.int32)
    src4d = _layout_edges(ei[0])
    dst4d = _layout_edges(ei[1])
    dst_deg = dst4d.reshape(NW, PASSES * CPPG, CHUNK)
    zeros1 = jnp.zeros((NPAD,), jnp.float32)
    zeros2 = jnp.zeros((NPAD, D), jnp.float32)
    xp = jnp.pad(x, ((0, NPAD - N), (0, 0)))
    b1r = b1.reshape(1, D)
    b2r = b2.reshape(1, D)
    blr = bl.reshape(1, 1)

    degp = _sc_degree(zeros1, dst_deg).reshape(NC, NPAD)
    y1 = _tc1(degp, xp, W1)
    p1 = _sc_propagate(y1, src4d, dst4d, zeros2)
    y2 = _tc2(degp, p1, y1, b1r, W2)
    p2 = _sc_propagate(y2, src4d, dst4d, zeros2)
    out = _tc3(degp, p2, y2, b2r, Wl, blr)
    return out[:N, 0]


# speedup vs baseline: 4.3049x; 1.1639x over previous
import functools

import jax
import jax.numpy as jnp
from jax import lax
from jax.experimental import pallas as pl
from jax.experimental.pallas import tpu as pltpu
from jax.experimental.pallas import tpu_sc as plsc

N = 10000
D = 128
E = 320000
NC = 2
NS = 16
NW = NC * NS
CHUNK = 64
PASSES = 2
CPP0 = 80
CPP1 = 78
CPPG = max(CPP0, CPP1) + 2
NPAD = 10240
RPT = NPAD // NS
BLK = 1024


def _mesh():
    return plsc.VectorSubcoreMesh(
        core_axis_name="c", subcore_axis_name="s", num_cores=NC, num_subcores=NS
    )


@functools.partial(
    pl.kernel,
    out_type=jax.ShapeDtypeStruct((NW, 1, RPT), jnp.float32),
    mesh=_mesh(),
    scratch_types=[
        pltpu.VMEM_SHARED((NPAD,), jnp.float32),
        pltpu.VMEM((PASSES * CPPG, CHUNK), jnp.int32),
        pltpu.VMEM((CHUNK,), jnp.float32),
        pltpu.SemaphoreType.DMA,
    ],
)
def _sc_degree(zeros1_hbm, dst_hbm, out_hbm, acc, dstv, ones_v, sem):
    cid = lax.axis_index("c")
    sid = lax.axis_index("s")
    wid = cid * NS + sid
    pltpu.sync_copy(zeros1_hbm.at[pl.ds(sid * RPT, RPT)],
                    acc.at[pl.ds(sid * RPT, RPT)])
    for i in range(CHUNK // 16):
        ones_v[pl.ds(i * 16, 16)] = jnp.ones((16,), jnp.float32)
    pltpu.sync_copy(dst_hbm.at[wid], dstv)
    plsc.subcore_barrier()
    descs = []
    for j in range(PASSES * CPPG):
        descs.append(pltpu.async_copy(ones_v, acc.at[dstv.at[j]], sem, add=True))
    for d in descs:
        d.wait()
    plsc.subcore_barrier()
    pltpu.sync_copy(acc.at[pl.ds(sid * RPT, RPT)], out_hbm.at[wid, 0])


@functools.partial(
    pl.kernel,
    out_type=jax.ShapeDtypeStruct((NC, NPAD, D), jnp.float32),
    mesh=_mesh(),
    scratch_types=[
        pltpu.VMEM_SHARED((NPAD, D), jnp.float32),
        pltpu.VMEM((CPPG, CHUNK), jnp.int32),
        pltpu.VMEM((CPPG, CHUNK), jnp.int32),
        pltpu.VMEM((CHUNK, D), jnp.float32),
        pltpu.VMEM((CHUNK, D), jnp.float32),
        pltpu.SemaphoreType.DMA,
        pltpu.SemaphoreType.DMA,
        pltpu.SemaphoreType.DMA,
    ],
)
def _sc_propagate(y_hbm, src_hbm, dst_hbm, zeros_hbm, out_hbm,
                  acc, srcv, dstv, buf0, buf1, gsem0, gsem1, ssem):
    cid = lax.axis_index("c")
    sid = lax.axis_index("s")
    wid = cid * NS + sid
    cpp = jnp.where(cid == 0, CPP0, CPP1)
    pltpu.sync_copy(zeros_hbm.at[pl.ds(sid * RPT, RPT)],
                    acc.at[pl.ds(sid * RPT, RPT)])
    plsc.subcore_barrier()

    for p in range(PASSES):
        pltpu.sync_copy(src_hbm.at[wid, p], srcv)
        pltpu.sync_copy(dst_hbm.at[wid, p], dstv)

        pltpu.async_copy(y_hbm.at[srcv.at[0]], buf0, gsem0)
        pltpu.async_copy(y_hbm.at[srcv.at[1]], buf1, gsem1)

        def body(i, carry):
            j0 = 2 * i
            pltpu.make_async_copy(y_hbm.at[srcv.at[j0]], buf0, gsem0).wait()
            pltpu.async_copy(buf0, acc.at[dstv.at[j0]], ssem, add=True).wait()
            pltpu.async_copy(y_hbm.at[srcv.at[j0 + 2]], buf0, gsem0)
            pltpu.make_async_copy(y_hbm.at[srcv.at[j0 + 1]], buf1, gsem1).wait()
            pltpu.async_copy(buf1, acc.at[dstv.at[j0 + 1]], ssem, add=True).wait()
            pltpu.async_copy(y_hbm.at[srcv.at[j0 + 3]], buf1, gsem1)
            return carry

        lax.fori_loop(0, cpp // 2, body, 0)
        pltpu.make_async_copy(y_hbm.at[srcv.at[cpp]], buf0, gsem0).wait()
        pltpu.make_async_copy(y_hbm.at[srcv.at[cpp + 1]], buf1, gsem1).wait()

    plsc.subcore_barrier()
    pltpu.sync_copy(acc.at[pl.ds(sid * RPT, RPT)],
                    out_hbm.at[cid, pl.ds(sid * RPT, RPT)])


def _dis(deg_ref):
    deg = deg_ref[0, :] + deg_ref[1, :] + 1.0
    return lax.rsqrt(deg)


def _tc1_body(deg_ref, x_ref, w_ref, y_ref):
    dis = _dis(deg_ref)
    y_ref[...] = dis[:, None] * jnp.dot(
        x_ref[...], w_ref[...], preferred_element_type=jnp.float32)


def _tc2_body(deg_ref, p_ref, y_ref, b_ref, w_ref, o_ref):
    dis = _dis(deg_ref)
    agg = p_ref[0] + p_ref[1] + y_ref[...]
    h = jnp.maximum(dis[:, None] * agg + b_ref[...], 0.0)
    o_ref[...] = dis[:, None] * jnp.dot(
        h, w_ref[...], preferred_element_type=jnp.float32)


def _tc3_body(deg_ref, p_ref, y_ref, b_ref, wl_ref, bl_ref, o_ref):
    dis = _dis(deg_ref)
    agg = p_ref[0] + p_ref[1] + y_ref[...]
    h = jnp.maximum(dis[:, None] * agg + b_ref[...], 0.0)
    o_ref[...] = jnp.dot(h, wl_ref[...],
                         preferred_element_type=jnp.float32) + bl_ref[...]


_GRID = NPAD // BLK
_deg_spec = pl.BlockSpec((NC, BLK), lambda i: (0, i))
_row_spec = pl.BlockSpec((BLK, D), lambda i: (i, 0))
_p_spec = pl.BlockSpec((NC, BLK, D), lambda i: (0, i, 0))
_w_spec = pl.BlockSpec((D, D), lambda i: (0, 0))
_b_spec = pl.BlockSpec((1, D), lambda i: (0, 0))
_wl_spec = pl.BlockSpec((D, 1), lambda i: (0, 0))
_bl_spec = pl.BlockSpec((1, 1), lambda i: (0, 0))

_tc1 = pl.pallas_call(
    _tc1_body, grid=(_GRID,),
    in_specs=[_deg_spec, _row_spec, _w_spec],
    out_specs=_row_spec,
    out_shape=jax.ShapeDtypeStruct((NPAD, D), jnp.float32),
)
_tc2 = pl.pallas_call(
    _tc2_body, grid=(_GRID,),
    in_specs=[_deg_spec, _p_spec, _row_spec, _b_spec, _w_spec],
    out_specs=_row_spec,
    out_shape=jax.ShapeDtypeStruct((NPAD, D), jnp.float32),
)
_tc3 = pl.pallas_call(
    _tc3_body, grid=(_GRID,),
    in_specs=[_deg_spec, _p_spec, _row_spec, _b_spec, _wl_spec, _bl_spec],
    out_specs=pl.BlockSpec((BLK, 1), lambda i: (i, 0)),
    out_shape=jax.ShapeDtypeStruct((NPAD, 1), jnp.float32),
)


def _layout_edges(idx):
    n0 = NS * PASSES * CPP0 * CHUNK
    n1 = NS * PASSES * CPP1 * CHUNK
    pad = N + (jnp.arange(n0 + n1 - E, dtype=jnp.int32) % (NPAD - N))
    idx = jnp.concatenate([idx, pad])
    part0 = idx[:n0].reshape(NS, PASSES, CPP0, CHUNK)
    part1 = idx[n0:].reshape(NS, PASSES, CPP1, CHUNK)
    out = N + (jnp.arange(NW * PASSES * CPPG * CHUNK, dtype=jnp.int32)
               % (NPAD - N)).reshape(NW, PASSES, CPPG, CHUNK)
    out = out.at[:NS, :, :CPP0].set(part0)
    out = out.at[NS:, :, :CPP1].set(part1)
    return out


def kernel(x, edge_index, W1, b1, W2, b2, Wl, bl):
    ei = edge_index.astype(jnp.int32)
    src4d = _layout_edges(ei[0])
    dst4d = _layout_edges(ei[1])
    dst_deg = dst4d.reshape(NW, PASSES * CPPG, CHUNK)
    zeros1 = jnp.zeros((NPAD,), jnp.float32)
    zeros2 = jnp.zeros((NPAD, D), jnp.float32)
    xp = jnp.pad(x, ((0, NPAD - N), (0, 0)))
    b1r = b1.reshape(1, D)
    b2r = b2.reshape(1, D)
    blr = bl.reshape(1, 1)

    degp = _sc_degree(zeros1, dst_deg).reshape(NC, NPAD)
    y1 = _tc1(degp, xp, W1)
    p1 = _sc_propagate(y1, src4d, dst4d, zeros2)
    y2 = _tc2(degp, p1, y1, b1r, W2)
    p2 = _sc_propagate(y2, src4d, dst4d, zeros2)
    out = _tc3(degp, p2, y2, b2r, Wl, blr)
    return out[:N, 0]
